# baseline (speedup 1.0000x reference)
import jax
import jax.numpy as jnp
from jax.experimental import pallas as pl

_B, _N, _M, _D = 4, 1024, 1024, 512
_K = 8
_SQ_UNDERFLOW = 11000.0


def _avig_body(vf_ref, af_ref, wv_ref, wa_ref, ev_ref, ea_ref):
    vf = vf_ref[0]
    af = af_ref[0]
    wv = wv_ref[...]
    wa = wa_ref[...]

    vm = jnp.dot(vf, wv, preferred_element_type=jnp.float32)
    am = jnp.dot(af, wa, preferred_element_type=jnp.float32)

    v2 = jnp.sum(vm * vm, axis=1, keepdims=True)
    a2 = jnp.sum(am * am, axis=1, keepdims=True)
    cross = jax.lax.dot_general(
        vm, am, (((1,), (1,)), ((), ())),
        preferred_element_type=jnp.float32)

    margin = 0.03125 * (jnp.max(v2) + jnp.max(a2))
    bound = jnp.min(v2) + jnp.min(a2) - 2.0 * jnp.max(cross)
    all_underflow = bound > _SQ_UNDERFLOW + margin

    @pl.when(all_underflow)
    def _fast():
        ev_ref[0] = vf[:_K, :]
        ea_ref[0] = af[:_K, :]

    @pl.when(jnp.logical_not(all_underflow))
    def _general():
        sq = jnp.maximum(v2 + a2.reshape(1, _M) - 2.0 * cross, 0.0)
        s = jnp.exp(-jnp.sqrt(sq))
        iota_n = jax.lax.broadcasted_iota(jnp.int32, (_N, _M), 0)
        wcols = []
        ss = s
        for _ in range(_K):
            vmax = jnp.max(ss, axis=0, keepdims=True)
            argm = jnp.min(jnp.where(ss == vmax, iota_n, _N), axis=0,
                           keepdims=True)
            sel = iota_n == argm
            wcols.append(
                jnp.sum(jnp.where(sel, 1.0, 0.0), axis=1, keepdims=True))
            ss = jnp.where(sel, -1.0, ss)

        w = jnp.concatenate(wcols, axis=1)
        inv_m = 1.0 / _M
        ev_ref[0] = jax.lax.dot_general(
            w, vf, (((0,), (0,)), ((), ())),
            precision=jax.lax.Precision.HIGHEST,
            preferred_element_type=jnp.float32) * inv_m
        ea_ref[0] = jax.lax.dot_general(
            w, af, (((0,), (0,)), ((), ())),
            precision=jax.lax.Precision.HIGHEST,
            preferred_element_type=jnp.float32) * inv_m


@jax.jit
def kernel(visual_features, audio_features, visual_weights, audio_weights):
    out_shape = jax.ShapeDtypeStruct((_B, _K, _D), jnp.float32)
    ev, ea = pl.pallas_call(
        _avig_body,
        grid=(_B,),
        in_specs=[
            pl.BlockSpec((1, _N, _D), lambda b: (b, 0, 0)),
            pl.BlockSpec((1, _M, _D), lambda b: (b, 0, 0)),
            pl.BlockSpec((_D, _D), lambda b: (0, 0)),
            pl.BlockSpec((_D, _D), lambda b: (0, 0)),
        ],
        out_specs=[
            pl.BlockSpec((1, _K, _D), lambda b: (b, 0, 0)),
            pl.BlockSpec((1, _K, _D), lambda b: (b, 0, 0)),
        ],
        out_shape=[out_shape, out_shape],
    )(visual_features, audio_features, visual_weights, audio_weights)
    return ev, ea


# speedup vs baseline: 4.9609x; 4.9609x over previous
import jax
import jax.numpy as jnp
from jax.experimental import pallas as pl

_B, _N, _M, _D = 4, 1024, 1024, 512
_K = 8
_SQ_UNDERFLOW = 11000.0


def _avig_body(vf_ref, af_ref, wv_ref, wa_ref, ev_ref, ea_ref):
    vf = vf_ref[0]
    af = af_ref[0]
    wv = wv_ref[...]
    wa = wa_ref[...]

    vm = jnp.dot(vf, wv, preferred_element_type=jnp.float32)
    am = jnp.dot(af, wa, preferred_element_type=jnp.float32)

    v2 = jnp.sum(vm * vm, axis=1, keepdims=True)
    a2 = jnp.sum(am * am, axis=1, keepdims=True)
    cross = jax.lax.dot_general(
        vm, am, (((1,), (1,)), ((), ())),
        preferred_element_type=jnp.float32)

    tmin = jnp.min(v2 - 2.0 * cross, axis=0, keepdims=True)
    sqmin = jnp.min(tmin + a2.reshape(1, _M))
    all_underflow = sqmin > _SQ_UNDERFLOW

    @pl.when(all_underflow)
    def _fast():
        ev_ref[0] = vf[:_K, :]
        ea_ref[0] = af[:_K, :]

    @pl.when(jnp.logical_not(all_underflow))
    def _general():
        sq = jnp.maximum(v2 + a2.reshape(1, _M) - 2.0 * cross, 0.0)
        s = jnp.exp(-jnp.sqrt(sq))
        iota_n = jax.lax.broadcasted_iota(jnp.int32, (_N, _M), 0)
        wcols = []
        ss = s
        for _ in range(_K):
            vmax = jnp.max(ss, axis=0, keepdims=True)
            argm = jnp.min(jnp.where(ss == vmax, iota_n, _N), axis=0,
                           keepdims=True)
            sel = iota_n == argm
            wcols.append(
                jnp.sum(jnp.where(sel, 1.0, 0.0), axis=1, keepdims=True))
            ss = jnp.where(sel, -1.0, ss)

        w = jnp.concatenate(wcols, axis=1)
        inv_m = 1.0 / _M
        ev_ref[0] = jax.lax.dot_general(
            w, vf, (((0,), (0,)), ((), ())),
            precision=jax.lax.Precision.HIGHEST,
            preferred_element_type=jnp.float32) * inv_m
        ea_ref[0] = jax.lax.dot_general(
            w, af, (((0,), (0,)), ((), ())),
            precision=jax.lax.Precision.HIGHEST,
            preferred_element_type=jnp.float32) * inv_m


@jax.jit
def kernel(visual_features, audio_features, visual_weights, audio_weights):
    out_shape = jax.ShapeDtypeStruct((_B, _K, _D), jnp.float32)
    ev, ea = pl.pallas_call(
        _avig_body,
        grid=(_B,),
        in_specs=[
            pl.BlockSpec((1, _N, _D), lambda b: (b, 0, 0)),
            pl.BlockSpec((1, _M, _D), lambda b: (b, 0, 0)),
            pl.BlockSpec((_D, _D), lambda b: (0, 0)),
            pl.BlockSpec((_D, _D), lambda b: (0, 0)),
        ],
        out_specs=[
            pl.BlockSpec((1, _K, _D), lambda b: (b, 0, 0)),
            pl.BlockSpec((1, _K, _D), lambda b: (b, 0, 0)),
        ],
        out_shape=[out_shape, out_shape],
    )(visual_features, audio_features, visual_weights, audio_weights)
    return ev, ea
